# Initial kernel scaffold; baseline (speedup 1.0000x reference)
import jax
import jax.numpy as jnp
from jax.experimental import pallas as pl


def kernel(points, params):
    raise NotImplementedError("write your pallas kernel here")


# speedup vs baseline: 1.5042x; 1.5042x over previous
import functools

import jax
import jax.numpy as jnp
from jax import lax
from jax.experimental import pallas as pl
from jax.experimental.pallas import tpu as pltpu

_N = 16384
_PATCH = 1024
_GRID = 0.02
_ENC_DEPTHS = (2, 2, 2, 4, 2)
_ENC_CH = (32, 64, 128, 256, 384)
_ENC_H = (2, 4, 8, 16, 24)
_DEC_DEPTHS = (2, 2, 2, 2)
_DEC_CH = (64, 64, 128, 256)
_DEC_H = (4, 4, 8, 16)

_INTERPRET = False


def _split3(a):
    a = a & jnp.uint32(0x3FF)
    a = (a | (a << 16)) & jnp.uint32(0x030000FF)
    a = (a | (a << 8)) & jnp.uint32(0x0300F00F)
    a = (a | (a << 4)) & jnp.uint32(0x030C30C3)
    a = (a | (a << 2)) & jnp.uint32(0x09249249)
    return a


def _morton(coord, grid):
    g = jnp.floor(coord / grid).astype(jnp.uint32)
    return _split3(g[:, 0]) | (_split3(g[:, 1]) << 1) | (_split3(g[:, 2]) << 2)


def _gather_rows(x, idx):
    return x[idx]


def _scatter_rows(y, idx, nrows):
    return jnp.zeros((nrows, y.shape[1]), y.dtype).at[idx].set(y)


def _ln(x, s, b):
    mu = jnp.mean(x, axis=-1, keepdims=True)
    var = jnp.mean((x - mu) ** 2, axis=-1, keepdims=True)
    return (x - mu) * jax.lax.rsqrt(var + 1e-5) * s + b


def _full2d(shape):
    return pl.BlockSpec(shape, lambda p: (0, 0))


@functools.lru_cache(None)
def _linear_call(M, K, N, RB):
    def body(x_ref, w_ref, b_ref, o_ref):
        o_ref[...] = (
            jnp.dot(x_ref[...], w_ref[...], preferred_element_type=jnp.float32)
            + b_ref[...]
        )

    return pl.pallas_call(
        body,
        grid=(M // RB,),
        in_specs=[
            pl.BlockSpec((RB, K), lambda p: (p, 0)),
            _full2d((K, N)),
            _full2d((1, N)),
        ],
        out_specs=pl.BlockSpec((RB, N), lambda p: (p, 0)),
        out_shape=jax.ShapeDtypeStruct((M, N), jnp.float32),
        compiler_params=pltpu.CompilerParams(
            dimension_semantics=("parallel",)
        ),
        interpret=_INTERPRET,
    )


def _linear(x, w, b):
    M, K = x.shape
    N = w.shape[1]
    RB = min(M, 2048)
    return _linear_call(M, K, N, RB)(x, w, b.reshape(1, N))


@functools.lru_cache(None)
def _block_call(M, C, H):
    d = C // H

    def body(x_ref, ln1s, ln1b, qkvw, qkvb, projw, projb,
             ln2s, ln2b, w1, b1, w2, b2, o_ref):
        x = x_ref[...]
        h = _ln(x, ln1s[...], ln1b[...])
        qkv = (
            jnp.dot(h, qkvw[...], preferred_element_type=jnp.float32)
            + qkvb[...]
        )
        outs = []
        for i in range(H):
            q = qkv[:, i * d:(i + 1) * d]
            k = qkv[:, C + i * d:C + (i + 1) * d]
            v = qkv[:, 2 * C + i * d:2 * C + (i + 1) * d]
            s = lax.dot_general(
                q, k, (((1,), (1,)), ((), ())),
                preferred_element_type=jnp.float32,
            ) * (d ** -0.5)
            a = jax.nn.softmax(s, axis=-1)
            outs.append(
                jnp.dot(a, v, preferred_element_type=jnp.float32))
        o = jnp.concatenate(outs, axis=1)
        x = x + jnp.dot(o, projw[...], preferred_element_type=jnp.float32) + projb[...]
        h2 = _ln(x, ln2s[...], ln2b[...])
        h2 = jax.nn.gelu(
            jnp.dot(h2, w1[...], preferred_element_type=jnp.float32) + b1[...])
        h2 = jnp.dot(h2, w2[...], preferred_element_type=jnp.float32) + b2[...]
        o_ref[...] = x + h2

    in_specs = [
        pl.BlockSpec((_PATCH, C), lambda p: (p, 0)),
        _full2d((1, C)), _full2d((1, C)),
        _full2d((C, 3 * C)), _full2d((1, 3 * C)),
        _full2d((C, C)), _full2d((1, C)),
        _full2d((1, C)), _full2d((1, C)),
        _full2d((C, 4 * C)), _full2d((1, 4 * C)),
        _full2d((4 * C, C)), _full2d((1, C)),
    ]
    return pl.pallas_call(
        body,
        grid=(M // _PATCH,),
        in_specs=in_specs,
        out_specs=pl.BlockSpec((_PATCH, C), lambda p: (p, 0)),
        out_shape=jax.ShapeDtypeStruct((M, C), jnp.float32),
        compiler_params=pltpu.CompilerParams(
            dimension_semantics=("parallel",)
        ),
        interpret=_INTERPRET,
    )


def _attn_block(x, p, H):
    M, C = x.shape
    return _block_call(M, C, H)(
        x,
        p['ln1_s'].reshape(1, C), p['ln1_b'].reshape(1, C),
        p['qkv_w'], p['qkv_b'].reshape(1, 3 * C),
        p['proj_w'], p['proj_b'].reshape(1, C),
        p['ln2_s'].reshape(1, C), p['ln2_b'].reshape(1, C),
        p['mlp_w1'], p['mlp_b1'].reshape(1, 4 * C),
        p['mlp_w2'], p['mlp_b2'].reshape(1, C),
    )


@functools.lru_cache(None)
def _down_call(M, C, C2):
    def body(x_ref, w_ref, b_ref, o_ref):
        y = (
            jnp.dot(x_ref[...], w_ref[...], preferred_element_type=jnp.float32)
            + b_ref[...]
        )
        o_ref[...] = jnp.max(y.reshape(_PATCH // 2, 2, C2), axis=1)

    return pl.pallas_call(
        body,
        grid=(M // _PATCH,),
        in_specs=[
            pl.BlockSpec((_PATCH, C), lambda p: (p, 0)),
            _full2d((C, C2)),
            _full2d((1, C2)),
        ],
        out_specs=pl.BlockSpec((_PATCH // 2, C2), lambda p: (p, 0)),
        out_shape=jax.ShapeDtypeStruct((M // 2, C2), jnp.float32),
        compiler_params=pltpu.CompilerParams(
            dimension_semantics=("parallel",)
        ),
        interpret=_INTERPRET,
    )


@functools.lru_cache(None)
def _merge_call(M, Ce, Cd, RB):
    def body(up_ref, skip_ref, w_ref, b_ref, o_ref):
        o_ref[...] = (
            up_ref[...]
            + jnp.dot(skip_ref[...], w_ref[...],
                      preferred_element_type=jnp.float32)
            + b_ref[...]
        )

    return pl.pallas_call(
        body,
        grid=(M // RB,),
        in_specs=[
            pl.BlockSpec((RB, Cd), lambda p: (p, 0)),
            pl.BlockSpec((RB, Ce), lambda p: (p, 0)),
            _full2d((Ce, Cd)),
            _full2d((1, Cd)),
        ],
        out_specs=pl.BlockSpec((RB, Cd), lambda p: (p, 0)),
        out_shape=jax.ShapeDtypeStruct((M, Cd), jnp.float32),
        compiler_params=pltpu.CompilerParams(
            dimension_semantics=("parallel",)
        ),
        interpret=_INTERPRET,
    )


@functools.lru_cache(None)
def _gmax_call(M, C):
    def body(x_ref, o_ref):
        o_ref[...] = jnp.max(x_ref[...], axis=0, keepdims=True)

    return pl.pallas_call(
        body,
        in_specs=[pl.BlockSpec((M, C), lambda: (0, 0))],
        out_specs=pl.BlockSpec((1, C), lambda: (0, 0)),
        out_shape=jax.ShapeDtypeStruct((1, C), jnp.float32),
        interpret=_INTERPRET,
    )


def kernel(points, params):
    flat = points.reshape(_N, 3)

    coord = flat
    orders = []
    for s in range(5):
        code = _morton(coord, _GRID * (2 ** s))
        order = jnp.argsort(code)
        orders.append(order)
        if s < 4:
            cs = _gather_rows(coord, order)
            coord = jnp.mean(cs.reshape(cs.shape[0] // 2, 2, 3), axis=1)

    x = _linear(flat, params['embed_w'], params['embed_b'])
    skips = []
    for s in range(5):
        x = _gather_rows(x, orders[s])
        for bp in params['enc'][s]['blocks']:
            x = _attn_block(x, bp, _ENC_H[s])
        skips.append(x)
        if s < 4:
            sp = params['enc'][s]
            x = _down_call(x.shape[0], _ENC_CH[s], _ENC_CH[s + 1])(
                x, sp['down_w'], sp['down_b'].reshape(1, -1))

    for s in range(3, -1, -1):
        dp = params['dec'][s]
        y = _linear(x, dp['up_w'], dp['up_b'])
        M = 2 * y.shape[0]
        up = _scatter_rows(
            jnp.concatenate([y, y], axis=0),
            jnp.concatenate([2 * orders[s + 1], 2 * orders[s + 1] + 1]),
            M)
        x = _merge_call(M, _ENC_CH[s], _DEC_CH[s], min(M, 2048))(
            up, skips[s], dp['skip_w'], dp['skip_b'].reshape(1, -1))
        for bp in dp['blocks']:
            x = _attn_block(x, bp, _DEC_H[s])

    global_feat = _gmax_call(_N, _DEC_CH[0])(x).reshape(_DEC_CH[0])
    per_point = _scatter_rows(x, orders[0], _N).reshape(1, _N, _DEC_CH[0])
    return per_point, global_feat[None]


# speedup vs baseline: 1.5752x; 1.0472x over previous
import functools

import jax
import jax.numpy as jnp
from jax import lax
from jax.experimental import pallas as pl
from jax.experimental.pallas import tpu as pltpu
from jax.experimental.pallas import tpu_sc as plsc

_N = 16384
_PATCH = 1024
_GRID = 0.02
_ENC_DEPTHS = (2, 2, 2, 4, 2)
_ENC_CH = (32, 64, 128, 256, 384)
_ENC_H = (2, 4, 8, 16, 24)
_DEC_DEPTHS = (2, 2, 2, 2)
_DEC_CH = (64, 64, 128, 256)
_DEC_H = (4, 4, 8, 16)

_INTERPRET = False


def _split3(a):
    a = a & jnp.uint32(0x3FF)
    a = (a | (a << 16)) & jnp.uint32(0x030000FF)
    a = (a | (a << 8)) & jnp.uint32(0x0300F00F)
    a = (a | (a << 4)) & jnp.uint32(0x030C30C3)
    a = (a | (a << 2)) & jnp.uint32(0x09249249)
    return a


def _morton(coord, grid):
    g = jnp.floor(coord / grid).astype(jnp.uint32)
    return _split3(g[:, 0]) | (_split3(g[:, 1]) << 1) | (_split3(g[:, 2]) << 2)


@functools.lru_cache(None)
def _sc_gather_call(V, D, B):
    info = plsc.get_sparse_core_info()
    NC, NS = info.num_cores, info.num_subcores
    NW = NC * NS
    b_per_w = B // NW
    mesh = plsc.VectorSubcoreMesh(core_axis_name="c", subcore_axis_name="s")

    @functools.partial(
        pl.kernel, mesh=mesh,
        out_type=jax.ShapeDtypeStruct((B, D), jnp.float32),
        scratch_types=[
            pltpu.VMEM((b_per_w,), jnp.int32),
            pltpu.VMEM((b_per_w, D), jnp.float32),
            pltpu.SemaphoreType.DMA,
        ],
    )
    def k(table_hbm, idx_hbm, out_hbm, idx_v, rows_v, sem):
        wid = lax.axis_index("s") * NC + lax.axis_index("c")
        base = wid * b_per_w
        pltpu.sync_copy(idx_hbm.at[pl.ds(base, b_per_w)], idx_v)
        pltpu.async_copy(table_hbm.at[idx_v], rows_v, sem).wait()
        pltpu.sync_copy(rows_v, out_hbm.at[pl.ds(base, b_per_w)])

    return k


def _gather_rows(x, idx):
    return _sc_gather_call(x.shape[0], x.shape[1], idx.shape[0])(x, idx)


@functools.lru_cache(None)
def _sc_scatter_call(Bs, D, nrows, dup):
    info = plsc.get_sparse_core_info()
    NC, NS = info.num_cores, info.num_subcores
    NW = NC * NS
    b_per_w = Bs // NW
    mesh = plsc.VectorSubcoreMesh(core_axis_name="c", subcore_axis_name="s")

    scratch = [
        pltpu.VMEM((b_per_w,), jnp.int32),
        pltpu.VMEM((b_per_w, D), jnp.float32),
        pltpu.SemaphoreType.DMA,
    ]
    if dup:
        scratch.insert(1, pltpu.VMEM((b_per_w,), jnp.int32))

    @functools.partial(
        pl.kernel, mesh=mesh,
        out_type=jax.ShapeDtypeStruct((nrows, D), jnp.float32),
        scratch_types=scratch,
    )
    def k(y_hbm, idx_hbm, out_hbm, idx_v, *rest):
        if dup:
            idx2_v, rows_v, sem = rest
        else:
            rows_v, sem = rest
        wid = lax.axis_index("s") * NC + lax.axis_index("c")
        base = wid * b_per_w
        pltpu.sync_copy(idx_hbm.at[pl.ds(base, b_per_w)], idx_v)
        pltpu.sync_copy(y_hbm.at[pl.ds(base, b_per_w)], rows_v)
        if dup:
            for j in range(b_per_w // 16):
                v = idx_v[pl.ds(j * 16, 16)]
                idx_v[pl.ds(j * 16, 16)] = v * 2
                idx2_v[pl.ds(j * 16, 16)] = v * 2 + 1
            pltpu.async_copy(rows_v, out_hbm.at[idx_v], sem).wait()
            pltpu.async_copy(rows_v, out_hbm.at[idx2_v], sem).wait()
        else:
            pltpu.async_copy(rows_v, out_hbm.at[idx_v], sem).wait()

    return k


def _scatter_rows(y, idx, nrows):
    return _sc_scatter_call(y.shape[0], y.shape[1], nrows, False)(y, idx)


def _scatter_rows_dup(y, idx):
    return _sc_scatter_call(y.shape[0], y.shape[1], 2 * y.shape[0], True)(y, idx)


def _ln(x, s, b):
    mu = jnp.mean(x, axis=-1, keepdims=True)
    var = jnp.mean((x - mu) ** 2, axis=-1, keepdims=True)
    return (x - mu) / jnp.sqrt(var + 1e-5) * s + b


def _full2d(shape):
    return pl.BlockSpec(shape, lambda p: (0, 0))


@functools.lru_cache(None)
def _linear_call(M, K, N, RB):
    def body(x_ref, w_ref, b_ref, o_ref):
        o_ref[...] = (
            jnp.dot(x_ref[...], w_ref[...], preferred_element_type=jnp.float32)
            + b_ref[...]
        )

    return pl.pallas_call(
        body,
        grid=(M // RB,),
        in_specs=[
            pl.BlockSpec((RB, K), lambda p: (p, 0)),
            _full2d((K, N)),
            _full2d((1, N)),
        ],
        out_specs=pl.BlockSpec((RB, N), lambda p: (p, 0)),
        out_shape=jax.ShapeDtypeStruct((M, N), jnp.float32),
        compiler_params=pltpu.CompilerParams(
            dimension_semantics=("parallel",)
        ),
        interpret=_INTERPRET,
    )


def _linear(x, w, b):
    M, K = x.shape
    N = w.shape[1]
    RB = min(M, 2048)
    return _linear_call(M, K, N, RB)(x, w, b.reshape(1, N))


@functools.lru_cache(None)
def _block_call(M, C, H, Cin, Cout):
    d = C // H

    def body(x_ref, ln1s, ln1b, qkvw, qkvb, projw, projb,
             ln2s, ln2b, w1, b1, w2, b2, o_ref):
        x = x_ref[...][:, :C]
        h = _ln(x, ln1s[...], ln1b[...])
        qkv = (
            jnp.dot(h, qkvw[...], preferred_element_type=jnp.float32)
            + qkvb[...]
        )
        outs = []
        for i in range(H):
            q = qkv[:, i * d:(i + 1) * d]
            k = qkv[:, C + i * d:C + (i + 1) * d]
            v = qkv[:, 2 * C + i * d:2 * C + (i + 1) * d]
            s = lax.dot_general(
                q, k, (((1,), (1,)), ((), ())),
                preferred_element_type=jnp.float32,
            ) * (d ** -0.5)
            a = jax.nn.softmax(s, axis=-1)
            outs.append(
                jnp.dot(a, v, preferred_element_type=jnp.float32))
        o = jnp.concatenate(outs, axis=1)
        x = x + jnp.dot(o, projw[...], preferred_element_type=jnp.float32) + projb[...]
        h2 = _ln(x, ln2s[...], ln2b[...])
        h2 = jax.nn.gelu(
            jnp.dot(h2, w1[...], preferred_element_type=jnp.float32) + b1[...])
        h2 = jnp.dot(h2, w2[...], preferred_element_type=jnp.float32) + b2[...]
        r = x + h2
        if Cout > C:
            r = jnp.concatenate(
                [r, jnp.zeros((_PATCH, Cout - C), jnp.float32)], axis=1)
        o_ref[...] = r

    in_specs = [
        pl.BlockSpec((_PATCH, Cin), lambda p: (p, 0)),
        _full2d((1, C)), _full2d((1, C)),
        _full2d((C, 3 * C)), _full2d((1, 3 * C)),
        _full2d((C, C)), _full2d((1, C)),
        _full2d((1, C)), _full2d((1, C)),
        _full2d((C, 4 * C)), _full2d((1, 4 * C)),
        _full2d((4 * C, C)), _full2d((1, C)),
    ]
    return pl.pallas_call(
        body,
        grid=(M // _PATCH,),
        in_specs=in_specs,
        out_specs=pl.BlockSpec((_PATCH, Cout), lambda p: (p, 0)),
        out_shape=jax.ShapeDtypeStruct((M, Cout), jnp.float32),
        compiler_params=pltpu.CompilerParams(
            dimension_semantics=("parallel",)
        ),
        interpret=_INTERPRET,
    )


def _attn_block(x, p, H, C, Cout=None):
    M = x.shape[0]
    Cout = C if Cout is None else Cout
    return _block_call(M, C, H, x.shape[1], Cout)(
        x,
        p['ln1_s'].reshape(1, C), p['ln1_b'].reshape(1, C),
        p['qkv_w'], p['qkv_b'].reshape(1, 3 * C),
        p['proj_w'], p['proj_b'].reshape(1, C),
        p['ln2_s'].reshape(1, C), p['ln2_b'].reshape(1, C),
        p['mlp_w1'], p['mlp_b1'].reshape(1, 4 * C),
        p['mlp_w2'], p['mlp_b2'].reshape(1, C),
    )


@functools.lru_cache(None)
def _down_call(M, C, C2):
    def body(x_ref, w_ref, b_ref, o_ref):
        y = (
            jnp.dot(x_ref[...], w_ref[...], preferred_element_type=jnp.float32)
            + b_ref[...]
        )
        o_ref[...] = jnp.max(y.reshape(_PATCH // 2, 2, C2), axis=1)

    return pl.pallas_call(
        body,
        grid=(M // _PATCH,),
        in_specs=[
            pl.BlockSpec((_PATCH, C), lambda p: (p, 0)),
            _full2d((C, C2)),
            _full2d((1, C2)),
        ],
        out_specs=pl.BlockSpec((_PATCH // 2, C2), lambda p: (p, 0)),
        out_shape=jax.ShapeDtypeStruct((M // 2, C2), jnp.float32),
        compiler_params=pltpu.CompilerParams(
            dimension_semantics=("parallel",)
        ),
        interpret=_INTERPRET,
    )


@functools.lru_cache(None)
def _merge_call(M, Ce, Cd, Cup, RB):
    def body(up_ref, skip_ref, w_ref, b_ref, o_ref):
        o_ref[...] = (
            up_ref[...][:, :Cd]
            + jnp.dot(skip_ref[...], w_ref[...],
                      preferred_element_type=jnp.float32)
            + b_ref[...]
        )

    return pl.pallas_call(
        body,
        grid=(M // RB,),
        in_specs=[
            pl.BlockSpec((RB, Cup), lambda p: (p, 0)),
            pl.BlockSpec((RB, Ce), lambda p: (p, 0)),
            _full2d((Ce, Cd)),
            _full2d((1, Cd)),
        ],
        out_specs=pl.BlockSpec((RB, Cd), lambda p: (p, 0)),
        out_shape=jax.ShapeDtypeStruct((M, Cd), jnp.float32),
        compiler_params=pltpu.CompilerParams(
            dimension_semantics=("parallel",)
        ),
        interpret=_INTERPRET,
    )


@functools.lru_cache(None)
def _gmax_call(M, C):
    def body(x_ref, o_ref):
        o_ref[...] = jnp.max(x_ref[...], axis=0, keepdims=True)

    return pl.pallas_call(
        body,
        in_specs=[pl.BlockSpec((M, C), lambda: (0, 0))],
        out_specs=pl.BlockSpec((1, C), lambda: (0, 0)),
        out_shape=jax.ShapeDtypeStruct((1, C), jnp.float32),
        interpret=_INTERPRET,
    )


def _pad_cols(w, n):
    return jnp.pad(w, ((0, 0), (0, n - w.shape[1])))


def kernel(points, params):
    flat = points.reshape(_N, 3)

    coord = jnp.pad(flat, ((0, 0), (0, 125)))
    orders = []
    for s in range(5):
        code = _morton(coord[:, :3], _GRID * (2 ** s))
        order = jnp.argsort(code)
        orders.append(order)
        if s < 4:
            cs = _gather_rows(coord, order)
            coord = jnp.mean(cs.reshape(cs.shape[0] // 2, 2, 128), axis=1)

    x = _linear(flat, _pad_cols(params['embed_w'], 128),
                _pad_cols(params['embed_b'][None], 128)[0])
    skips = []
    for s in range(5):
        C, H = _ENC_CH[s], _ENC_H[s]
        x = _gather_rows(x, orders[s])
        for bp in params['enc'][s]['blocks']:
            x = _attn_block(x, bp, H, C)
        skips.append(x)
        if s < 4:
            sp = params['enc'][s]
            C2 = _ENC_CH[s + 1]
            C2p = max(C2, 128)
            x = _down_call(x.shape[0], C, C2p)(
                x, _pad_cols(sp['down_w'], C2p),
                _pad_cols(sp['down_b'][None], C2p))

    for s in range(3, -1, -1):
        dp = params['dec'][s]
        Cd = _DEC_CH[s]
        Cdp = max(Cd, 128)
        y = _linear(x, _pad_cols(dp['up_w'], Cdp),
                    _pad_cols(dp['up_b'][None], Cdp)[0])
        M = 2 * y.shape[0]
        up = _scatter_rows_dup(y, orders[s + 1])
        x = _merge_call(M, _ENC_CH[s], Cd, Cdp, min(M, 2048))(
            up, skips[s], dp['skip_w'], dp['skip_b'].reshape(1, -1))
        for bp in dp['blocks'][:-1]:
            x = _attn_block(x, bp, _DEC_H[s], Cd)
        x = _attn_block(x, dp['blocks'][-1], _DEC_H[s], Cd,
                        Cout=128 if s == 0 else Cd)

    global_feat = _gmax_call(_N, 128)(x).reshape(128)[: _DEC_CH[0]]
    per_point = _scatter_rows(x, orders[0], _N)[:, : _DEC_CH[0]]
    per_point = per_point.reshape(1, _N, _DEC_CH[0])
    return per_point, global_feat[None]


# speedup vs baseline: 1.6210x; 1.0291x over previous
import functools

import jax
import jax.numpy as jnp
from jax import lax
from jax.experimental import pallas as pl
from jax.experimental.pallas import tpu as pltpu
from jax.experimental.pallas import tpu_sc as plsc

_N = 16384
_PATCH = 1024
_GRID = 0.02
_ENC_DEPTHS = (2, 2, 2, 4, 2)
_ENC_CH = (32, 64, 128, 256, 384)
_ENC_H = (2, 4, 8, 16, 24)
_DEC_DEPTHS = (2, 2, 2, 2)
_DEC_CH = (64, 64, 128, 256)
_DEC_H = (4, 4, 8, 16)

_INTERPRET = False


def _split3(a):
    a = a & jnp.uint32(0x3FF)
    a = (a | (a << 16)) & jnp.uint32(0x030000FF)
    a = (a | (a << 8)) & jnp.uint32(0x0300F00F)
    a = (a | (a << 4)) & jnp.uint32(0x030C30C3)
    a = (a | (a << 2)) & jnp.uint32(0x09249249)
    return a


def _morton(coord, grid):
    g = jnp.floor(coord / grid).astype(jnp.uint32)
    return _split3(g[:, 0]) | (_split3(g[:, 1]) << 1) | (_split3(g[:, 2]) << 2)


@functools.lru_cache(None)
def _sc_gather_call(V, D, B):
    info = plsc.get_sparse_core_info()
    NC, NS = info.num_cores, info.num_subcores
    NW = NC * NS
    b_per_w = B // NW
    mesh = plsc.VectorSubcoreMesh(core_axis_name="c", subcore_axis_name="s")

    @functools.partial(
        pl.kernel, mesh=mesh,
        out_type=jax.ShapeDtypeStruct((B, D), jnp.float32),
        scratch_types=[
            pltpu.VMEM((b_per_w,), jnp.int32),
            pltpu.VMEM((b_per_w, D), jnp.float32),
            pltpu.SemaphoreType.DMA,
        ],
    )
    def k(table_hbm, idx_hbm, out_hbm, idx_v, rows_v, sem):
        wid = lax.axis_index("s") * NC + lax.axis_index("c")
        base = wid * b_per_w
        pltpu.sync_copy(idx_hbm.at[pl.ds(base, b_per_w)], idx_v)
        pltpu.async_copy(table_hbm.at[idx_v], rows_v, sem).wait()
        pltpu.sync_copy(rows_v, out_hbm.at[pl.ds(base, b_per_w)])

    return k


def _gather_rows(x, idx):
    return _sc_gather_call(x.shape[0], x.shape[1], idx.shape[0])(x, idx)


@functools.lru_cache(None)
def _sc_scatter_call(Bs, D, nrows, dup):
    info = plsc.get_sparse_core_info()
    NC, NS = info.num_cores, info.num_subcores
    NW = NC * NS
    b_per_w = Bs // NW
    mesh = plsc.VectorSubcoreMesh(core_axis_name="c", subcore_axis_name="s")

    scratch = [
        pltpu.VMEM((b_per_w,), jnp.int32),
        pltpu.VMEM((b_per_w, D), jnp.float32),
        pltpu.SemaphoreType.DMA,
    ]
    if dup:
        scratch.insert(1, pltpu.VMEM((b_per_w,), jnp.int32))

    @functools.partial(
        pl.kernel, mesh=mesh,
        out_type=jax.ShapeDtypeStruct((nrows, D), jnp.float32),
        scratch_types=scratch,
    )
    def k(y_hbm, idx_hbm, out_hbm, idx_v, *rest):
        if dup:
            idx2_v, rows_v, sem = rest
        else:
            rows_v, sem = rest
        wid = lax.axis_index("s") * NC + lax.axis_index("c")
        base = wid * b_per_w
        pltpu.sync_copy(idx_hbm.at[pl.ds(base, b_per_w)], idx_v)
        pltpu.sync_copy(y_hbm.at[pl.ds(base, b_per_w)], rows_v)
        if dup:
            for j in range(b_per_w // 16):
                v = idx_v[pl.ds(j * 16, 16)]
                idx_v[pl.ds(j * 16, 16)] = v * 2
                idx2_v[pl.ds(j * 16, 16)] = v * 2 + 1
            pltpu.async_copy(rows_v, out_hbm.at[idx_v], sem).wait()
            pltpu.async_copy(rows_v, out_hbm.at[idx2_v], sem).wait()
        else:
            pltpu.async_copy(rows_v, out_hbm.at[idx_v], sem).wait()

    return k


def _scatter_rows(y, idx, nrows):
    return _sc_scatter_call(y.shape[0], y.shape[1], nrows, False)(y, idx)


def _scatter_rows_dup(y, idx):
    return _sc_scatter_call(y.shape[0], y.shape[1], 2 * y.shape[0], True)(y, idx)


def _ln(x, s, b):
    mu = jnp.mean(x, axis=-1, keepdims=True)
    var = jnp.mean((x - mu) ** 2, axis=-1, keepdims=True)
    return (x - mu) / jnp.sqrt(var + 1e-5) * s + b


def _full2d(shape):
    return pl.BlockSpec(shape, lambda p: (0, 0))


@functools.lru_cache(None)
def _linear_call(M, K, N, RB):
    def body(x_ref, w_ref, b_ref, o_ref):
        o_ref[...] = (
            jnp.dot(x_ref[...], w_ref[...], preferred_element_type=jnp.float32)
            + b_ref[...]
        )

    return pl.pallas_call(
        body,
        grid=(M // RB,),
        in_specs=[
            pl.BlockSpec((RB, K), lambda p: (p, 0)),
            _full2d((K, N)),
            _full2d((1, N)),
        ],
        out_specs=pl.BlockSpec((RB, N), lambda p: (p, 0)),
        out_shape=jax.ShapeDtypeStruct((M, N), jnp.float32),
        compiler_params=pltpu.CompilerParams(
            dimension_semantics=("parallel",)
        ),
        interpret=_INTERPRET,
    )


def _linear(x, w, b):
    M, K = x.shape
    N = w.shape[1]
    RB = min(M, 2048)
    return _linear_call(M, K, N, RB)(x, w, b.reshape(1, N))


def _block_math(x, p12, C, H):
    ln1s, ln1b, qkvw, qkvb, projw, projb, ln2s, ln2b, w1, b1, w2, b2 = p12
    d = C // H
    h = _ln(x, ln1s, ln1b)
    qkv = jnp.dot(h, qkvw, preferred_element_type=jnp.float32) + qkvb
    outs = []
    for i in range(H):
        q = qkv[:, i * d:(i + 1) * d]
        k = qkv[:, C + i * d:C + (i + 1) * d]
        v = qkv[:, 2 * C + i * d:2 * C + (i + 1) * d]
        s = lax.dot_general(
            q, k, (((1,), (1,)), ((), ())),
            preferred_element_type=jnp.float32,
        ) * (d ** -0.5)
        a = jax.nn.softmax(s, axis=-1)
        outs.append(jnp.dot(a, v, preferred_element_type=jnp.float32))
    o = jnp.concatenate(outs, axis=1)
    x = x + jnp.dot(o, projw, preferred_element_type=jnp.float32) + projb
    h2 = _ln(x, ln2s, ln2b)
    h2 = jax.nn.gelu(
        jnp.dot(h2, w1, preferred_element_type=jnp.float32) + b1)
    h2 = jnp.dot(h2, w2, preferred_element_type=jnp.float32) + b2
    return x + h2


@functools.lru_cache(None)
def _stage_call(M, C, H, Cin, Cout, depth):

    def body(x_ref, *refs):
        o_ref = refs[-1]
        x = x_ref[...][:, :C]
        for bi in range(depth):
            x = _block_math(x, [r[...] for r in refs[bi * 12:bi * 12 + 12]],
                            C, H)
        if Cout > C:
            x = jnp.concatenate(
                [x, jnp.zeros((_PATCH, Cout - C), jnp.float32)], axis=1)
        o_ref[...] = x

    wspecs = [
        _full2d((1, C)), _full2d((1, C)),
        _full2d((C, 3 * C)), _full2d((1, 3 * C)),
        _full2d((C, C)), _full2d((1, C)),
        _full2d((1, C)), _full2d((1, C)),
        _full2d((C, 4 * C)), _full2d((1, 4 * C)),
        _full2d((4 * C, C)), _full2d((1, C)),
    ]
    in_specs = [pl.BlockSpec((_PATCH, Cin), lambda p: (p, 0))] + wspecs * depth
    return pl.pallas_call(
        body,
        grid=(M // _PATCH,),
        in_specs=in_specs,
        out_specs=pl.BlockSpec((_PATCH, Cout), lambda p: (p, 0)),
        out_shape=jax.ShapeDtypeStruct((M, Cout), jnp.float32),
        compiler_params=pltpu.CompilerParams(
            dimension_semantics=("parallel",)
        ),
        interpret=_INTERPRET,
    )


def _flat_block_params(p, C):
    return [
        p['ln1_s'].reshape(1, C), p['ln1_b'].reshape(1, C),
        p['qkv_w'], p['qkv_b'].reshape(1, 3 * C),
        p['proj_w'], p['proj_b'].reshape(1, C),
        p['ln2_s'].reshape(1, C), p['ln2_b'].reshape(1, C),
        p['mlp_w1'], p['mlp_b1'].reshape(1, 4 * C),
        p['mlp_w2'], p['mlp_b2'].reshape(1, C),
    ]


def _attn_stage(x, bps, H, C, Cout=None, max_depth=2):
    M = x.shape[0]
    Cout = C if Cout is None else Cout
    i = 0
    while i < len(bps):
        chunk = bps[i:i + max_depth]
        last = i + len(chunk) == len(bps)
        co = Cout if last else C
        args = [x]
        for bp in chunk:
            args += _flat_block_params(bp, C)
        x = _stage_call(M, C, H, x.shape[1], co, len(chunk))(*args)
        i += len(chunk)
    return x


@functools.lru_cache(None)
def _down_call(M, C, C2):
    def body(x_ref, w_ref, b_ref, o_ref):
        y = (
            jnp.dot(x_ref[...], w_ref[...], preferred_element_type=jnp.float32)
            + b_ref[...]
        )
        o_ref[...] = jnp.max(y.reshape(_PATCH // 2, 2, C2), axis=1)

    return pl.pallas_call(
        body,
        grid=(M // _PATCH,),
        in_specs=[
            pl.BlockSpec((_PATCH, C), lambda p: (p, 0)),
            _full2d((C, C2)),
            _full2d((1, C2)),
        ],
        out_specs=pl.BlockSpec((_PATCH // 2, C2), lambda p: (p, 0)),
        out_shape=jax.ShapeDtypeStruct((M // 2, C2), jnp.float32),
        compiler_params=pltpu.CompilerParams(
            dimension_semantics=("parallel",)
        ),
        interpret=_INTERPRET,
    )


@functools.lru_cache(None)
def _merge_call(M, Ce, Cd, Cup, RB):
    def body(up_ref, skip_ref, w_ref, b_ref, o_ref):
        o_ref[...] = (
            up_ref[...][:, :Cd]
            + jnp.dot(skip_ref[...], w_ref[...],
                      preferred_element_type=jnp.float32)
            + b_ref[...]
        )

    return pl.pallas_call(
        body,
        grid=(M // RB,),
        in_specs=[
            pl.BlockSpec((RB, Cup), lambda p: (p, 0)),
            pl.BlockSpec((RB, Ce), lambda p: (p, 0)),
            _full2d((Ce, Cd)),
            _full2d((1, Cd)),
        ],
        out_specs=pl.BlockSpec((RB, Cd), lambda p: (p, 0)),
        out_shape=jax.ShapeDtypeStruct((M, Cd), jnp.float32),
        compiler_params=pltpu.CompilerParams(
            dimension_semantics=("parallel",)
        ),
        interpret=_INTERPRET,
    )


@functools.lru_cache(None)
def _gmax_call(M, C):
    def body(x_ref, o_ref):
        o_ref[...] = jnp.max(x_ref[...], axis=0, keepdims=True)

    return pl.pallas_call(
        body,
        in_specs=[pl.BlockSpec((M, C), lambda: (0, 0))],
        out_specs=pl.BlockSpec((1, C), lambda: (0, 0)),
        out_shape=jax.ShapeDtypeStruct((1, C), jnp.float32),
        interpret=_INTERPRET,
    )


def _pad_cols(w, n):
    return jnp.pad(w, ((0, 0), (0, n - w.shape[1])))


def kernel(points, params):
    flat = points.reshape(_N, 3)

    coord = jnp.pad(flat, ((0, 0), (0, 125)))
    orders = []
    for s in range(5):
        code = _morton(coord[:, :3], _GRID * (2 ** s))
        order = jnp.argsort(code)
        orders.append(order)
        if s < 4:
            cs = _gather_rows(coord, order)
            coord = jnp.mean(cs.reshape(cs.shape[0] // 2, 2, 128), axis=1)

    x = _linear(flat, _pad_cols(params['embed_w'], 128),
                _pad_cols(params['embed_b'][None], 128)[0])
    skips = []
    for s in range(5):
        C, H = _ENC_CH[s], _ENC_H[s]
        x = _gather_rows(x, orders[s])
        x = _attn_stage(x, params['enc'][s]['blocks'], H, C)
        skips.append(x)
        if s < 4:
            sp = params['enc'][s]
            C2 = _ENC_CH[s + 1]
            C2p = max(C2, 128)
            x = _down_call(x.shape[0], C, C2p)(
                x, _pad_cols(sp['down_w'], C2p),
                _pad_cols(sp['down_b'][None], C2p))

    for s in range(3, -1, -1):
        dp = params['dec'][s]
        Cd = _DEC_CH[s]
        Cdp = max(Cd, 128)
        y = _linear(x, _pad_cols(dp['up_w'], Cdp),
                    _pad_cols(dp['up_b'][None], Cdp)[0])
        M = 2 * y.shape[0]
        up = _scatter_rows_dup(y, orders[s + 1])
        x = _merge_call(M, _ENC_CH[s], Cd, Cdp, min(M, 2048))(
            up, skips[s], dp['skip_w'], dp['skip_b'].reshape(1, -1))
        x = _attn_stage(x, dp['blocks'], _DEC_H[s], Cd,
                        Cout=128 if s == 0 else Cd)

    global_feat = _gmax_call(_N, 128)(x).reshape(128)[: _DEC_CH[0]]
    per_point = _scatter_rows(x, orders[0], _N)[:, : _DEC_CH[0]]
    per_point = per_point.reshape(1, _N, _DEC_CH[0])
    return per_point, global_feat[None]
